# baseline (speedup 1.0000x reference)
import functools

import jax
import jax.numpy as jnp
from jax import lax
from jax.experimental import pallas as pl
from jax.experimental.pallas import tpu as pltpu
from jax.experimental.pallas import tpu_sc as plsc

VOCAB = 100000
EMBED_DIM = 64
NUM_TOKENS = 819200

_NC = 2
_NS = 16
_NW = _NC * _NS
_B_PER_W = NUM_TOKENS // _NW
_CHUNK = 256
_JPC = _CHUNK // 128
_N_CHUNKS = _B_PER_W // _CHUNK
_NIT = EMBED_DIM // 8
_NJT = NUM_TOKENS // 128
_TBUF = _JPC * 8 * 128

_mesh = plsc.VectorSubcoreMesh(core_axis_name="c", subcore_axis_name="s")


@functools.partial(
    pl.kernel,
    mesh=_mesh,
    out_type=jax.ShapeDtypeStruct((_NIT, _NJT // _JPC, _JPC * 8 * 128),
                                   jnp.float32),
    scratch_types=[
        pltpu.VMEM((_CHUNK,), jnp.int32),
        pltpu.VMEM((_CHUNK,), jnp.int32),
        pltpu.VMEM((_CHUNK, EMBED_DIM), jnp.float32),
        pltpu.VMEM((_CHUNK, EMBED_DIM), jnp.float32),
        pltpu.VMEM((_NIT * _TBUF,), jnp.float32),
        pltpu.VMEM((_NIT * _TBUF,), jnp.float32),
        pltpu.SemaphoreType.DMA,
        pltpu.SemaphoreType.DMA,
        pltpu.SemaphoreType.DMA,
        pltpu.SemaphoreType.DMA,
    ],
    compiler_params=pltpu.CompilerParams(use_tc_tiling_on_sc=False,
                                         needs_layout_passes=False),
)
def _gather_kernel(idx_hbm, table_hbm, out_hbm,
                   idx_v0, idx_v1, rows_v0, rows_v1, tbuf0, tbuf1,
                   sem_g0, sem_g1, sem_w0, sem_w1):
    wid = lax.axis_index("s") * _NC + lax.axis_index("c")
    base = wid * _B_PER_W
    jbase = wid * (_B_PER_W // 128)
    bufs = ((idx_v0, rows_v0, tbuf0, sem_g0, sem_w0),
            (idx_v1, rows_v1, tbuf1, sem_g1, sem_w1))
    iota = lax.iota(jnp.int32, 16)

    rots = []
    vparts = []
    for r in range(16):
        rr = (iota + r) & 15
        rots.append(rr)
        vparts.append((rr >> 3) * (_JPC * 1024) + (rr & 7) * 128 + iota)

    def stage_and_gather(b, i):
        idx_v, rows_v, _, sem_g, _ = bufs[b]
        off = base + i * _CHUNK
        pltpu.sync_copy(idx_hbm.at[pl.ds(off, _CHUNK)], idx_v)
        pltpu.async_copy(table_hbm.at[idx_v], rows_v, sem_g)

    def wait_gather(b):
        idx_v, rows_v, _, sem_g, _ = bufs[b]
        pltpu.make_async_copy(table_hbm.at[idx_v], rows_v, sem_g).wait()

    def transpose(b):
        _, rows_v, tbuf, _, _ = bufs[b]

        def tloop(k, carry):
            jl = k // 8
            lg = k % 8
            row_ids = jl * 128 + lg * 16 + iota
            sb_tok = jl * 1024 + lg * 16
            for fblk in range(EMBED_DIM // 16):
                sbase = fblk * (2 * _JPC * 1024) + sb_tok
                for r in range(16):
                    vec = plsc.load_gather(
                        rows_v, [row_ids, rots[r] + (fblk * 16)])
                    plsc.store_scatter(tbuf, [vparts[r] + sbase], vec)
            return carry

        lax.fori_loop(0, _JPC * 8, tloop, 0, unroll=8)

    def start_writes(b, i):
        _, _, tbuf, _, sem_w = bufs[b]
        c0 = jbase // _JPC + i
        for it in range(_NIT):
            pltpu.async_copy(tbuf.at[pl.ds(it * _TBUF, _TBUF)],
                             out_hbm.at[it, c0], sem_w)

    def wait_writes(b, i):
        _, _, tbuf, _, sem_w = bufs[b]
        c0 = jbase // _JPC + i
        for it in range(_NIT):
            pltpu.make_async_copy(tbuf.at[pl.ds(it * _TBUF, _TBUF)],
                                  out_hbm.at[it, c0], sem_w).wait()

    stage_and_gather(0, 0)
    stage_and_gather(1, 1)

    def body(g, carry):
        for b in range(2):
            i = 2 * g + b
            wait_gather(b)

            @pl.when(i >= 2)
            def _():
                wait_writes(b, i - 2)

            transpose(b)
            start_writes(b, i)

            @pl.when(i + 2 < _N_CHUNKS)
            def _():
                stage_and_gather(b, i + 2)

        return carry

    lax.fori_loop(0, _N_CHUNKS // 2, body, 0)

    for b in range(2):
        wait_writes(b, _N_CHUNKS - 2 + b)


def kernel(word_sequence, table):
    idx = word_sequence.astype(jnp.int32)
    out3 = _gather_kernel(idx, table)
    out4 = out3.reshape(_NIT, _NJT, 8, 128)
    return out4.transpose(1, 3, 0, 2).reshape(NUM_TOKENS, EMBED_DIM)


# speedup vs baseline: 1.6729x; 1.6729x over previous
import functools

import jax
import jax.numpy as jnp
from jax import lax
from jax.experimental import pallas as pl
from jax.experimental.pallas import tpu as pltpu
from jax.experimental.pallas import tpu_sc as plsc

VOCAB = 100000
EMBED_DIM = 64
NUM_TOKENS = 819200

_NC = 2
_NS = 16
_NW = _NC * _NS
_B_PER_W = NUM_TOKENS // _NW
_CHUNK = 256
_JPC = _CHUNK // 128
_N_CHUNKS = _B_PER_W // _CHUNK
_NIT = EMBED_DIM // 8
_NJT = NUM_TOKENS // 128
_TBUF = _JPC * 8 * 128

_mesh = plsc.VectorSubcoreMesh(core_axis_name="c", subcore_axis_name="s")


@functools.partial(
    pl.kernel,
    mesh=_mesh,
    out_type=jax.ShapeDtypeStruct((_NIT, _NJT // _JPC, _JPC * 8 * 128),
                                   jnp.float32),
    scratch_types=[
        pltpu.VMEM((_CHUNK,), jnp.int32),
        pltpu.VMEM((_CHUNK,), jnp.int32),
        pltpu.VMEM((_CHUNK, EMBED_DIM), jnp.float32),
        pltpu.VMEM((_CHUNK, EMBED_DIM), jnp.float32),
        pltpu.VMEM((_NIT * _TBUF,), jnp.float32),
        pltpu.VMEM((_NIT * _TBUF,), jnp.float32),
        pltpu.SemaphoreType.DMA,
        pltpu.SemaphoreType.DMA,
        pltpu.SemaphoreType.DMA,
        pltpu.SemaphoreType.DMA,
    ],
    compiler_params=pltpu.CompilerParams(use_tc_tiling_on_sc=False,
                                         needs_layout_passes=False),
)
def _gather_kernel(idx_hbm, table_hbm, out_hbm,
                   idx_v0, idx_v1, rows_v0, rows_v1, tbuf0, tbuf1,
                   sem_g0, sem_g1, sem_w0, sem_w1):
    wid = lax.axis_index("s") * _NC + lax.axis_index("c")
    base = wid * _B_PER_W
    jbase = wid * (_B_PER_W // 128)
    bufs = ((idx_v0, rows_v0, tbuf0, sem_g0, sem_w0),
            (idx_v1, rows_v1, tbuf1, sem_g1, sem_w1))
    iota = lax.iota(jnp.int32, 16)

    rots = []
    vparts = []
    for r in range(16):
        rr = (iota + r) & 15
        rots.append(rr)
        vparts.append((rr >> 3) * (_JPC * 1024) + (rr & 7) * 128 + iota)

    def stage_and_gather(b, i):
        idx_v, rows_v, _, sem_g, _ = bufs[b]
        off = base + i * _CHUNK
        pltpu.sync_copy(idx_hbm.at[pl.ds(off, _CHUNK)], idx_v)
        pltpu.async_copy(table_hbm.at[idx_v], rows_v, sem_g)

    def wait_gather(b):
        idx_v, rows_v, _, sem_g, _ = bufs[b]
        pltpu.make_async_copy(table_hbm.at[idx_v], rows_v, sem_g).wait()

    def transpose(b):
        _, rows_v, tbuf, _, _ = bufs[b]

        def tloop(k, carry):
            jl = k // 8
            lg = k % 8
            row_ids = jl * 128 + lg * 16 + iota
            sb_tok = jl * 1024 + lg * 16
            for fblk in range(EMBED_DIM // 16):
                sbase = fblk * (2 * _JPC * 1024) + sb_tok
                for r in range(16):
                    vec = plsc.load_gather(
                        rows_v, [row_ids, rots[r] + (fblk * 16)])
                    plsc.store_scatter(tbuf, [vparts[r] + sbase], vec)
            return carry

        lax.fori_loop(0, _JPC * 8, tloop, 0, unroll=4)

    def start_writes(b, i):
        _, _, tbuf, _, sem_w = bufs[b]
        c0 = jbase // _JPC + i
        for it in range(_NIT):
            pltpu.async_copy(tbuf.at[pl.ds(it * _TBUF, _TBUF)],
                             out_hbm.at[it, c0], sem_w)

    def wait_writes(b, i):
        _, _, tbuf, _, sem_w = bufs[b]
        c0 = jbase // _JPC + i
        for it in range(_NIT):
            pltpu.make_async_copy(tbuf.at[pl.ds(it * _TBUF, _TBUF)],
                                  out_hbm.at[it, c0], sem_w).wait()

    stage_and_gather(0, 0)
    stage_and_gather(1, 1)

    def body(g, carry):
        for b in range(2):
            i = 2 * g + b
            wait_gather(b)

            @pl.when(i >= 2)
            def _():
                wait_writes(b, i - 2)

            transpose(b)
            start_writes(b, i)

            @pl.when(i + 2 < _N_CHUNKS)
            def _():
                stage_and_gather(b, i + 2)

        return carry

    lax.fori_loop(0, _N_CHUNKS // 2, body, 0)

    for b in range(2):
        wait_writes(b, _N_CHUNKS - 2 + b)


def kernel(word_sequence, table):
    idx = word_sequence.astype(jnp.int32)
    out3 = _gather_kernel(idx, table)
    out4 = out3.reshape(_NIT, _NJT, 8, 128)
    return out4.transpose(1, 3, 0, 2).reshape(NUM_TOKENS, EMBED_DIM)
